# baseline (device time: 21651 ns/iter reference)
import jax
import jax.numpy as jnp
from jax import lax
from jax.experimental import pallas as pl
from jax.experimental.pallas import tpu as pltpu

M, D = 512, 512
HALF = M // 2


def kernel(partial, resid, gamma):
    def body(partial_ref, resid_ref, gamma_ref, out_ref, recv_ref, sems):
        my_x = lax.axis_index("x")
        my_y = lax.axis_index("y")
        y_nbr = (my_x, 1 - my_y)
        x_nbr = (1 - my_x, my_y)

        barrier = pltpu.get_barrier_semaphore()
        for nbr in (y_nbr, x_nbr):
            pl.semaphore_signal(
                barrier, inc=1, device_id=nbr,
                device_id_type=pl.DeviceIdType.MESH,
            )
        pl.semaphore_wait(barrier, 2)

        row0 = my_x * HALF

        rdma_a = pltpu.make_async_remote_copy(
            src_ref=partial_ref.at[0, pl.ds(row0, HALF), :],
            dst_ref=recv_ref,
            send_sem=sems.at[0],
            recv_sem=sems.at[1],
            device_id=y_nbr,
            device_id_type=pl.DeviceIdType.MESH,
        )
        rdma_a.start()
        rdma_a.wait()

        y = (
            partial_ref[0, pl.ds(row0, HALF), :]
            + recv_ref[:, :]
            + resid_ref[pl.ds(row0, HALF), :]
        )
        rms = jnp.sqrt(jnp.mean(y * y, axis=-1, keepdims=True) + 1e-6)
        out_ref[pl.ds(row0, HALF), :] = y / rms * gamma_ref[0, :][None, :]

        rdma_b = pltpu.make_async_remote_copy(
            src_ref=out_ref.at[pl.ds(row0, HALF), :],
            dst_ref=out_ref.at[pl.ds(row0, HALF), :],
            send_sem=sems.at[2],
            recv_sem=sems.at[3],
            device_id=x_nbr,
            device_id_type=pl.DeviceIdType.MESH,
        )
        rdma_b.start()
        rdma_b.wait()

    return pl.pallas_call(
        body,
        out_shape=jax.ShapeDtypeStruct((M, D), jnp.float32),
        in_specs=[
            pl.BlockSpec(memory_space=pltpu.VMEM),
            pl.BlockSpec(memory_space=pltpu.VMEM),
            pl.BlockSpec(memory_space=pltpu.VMEM),
        ],
        out_specs=pl.BlockSpec(memory_space=pltpu.VMEM),
        scratch_shapes=[
            pltpu.VMEM((HALF, D), jnp.float32),
            pltpu.SemaphoreType.DMA((4,)),
        ],
        compiler_params=pltpu.CompilerParams(collective_id=0),
    )(partial, resid, gamma.reshape(1, D))


# device time: 16912 ns/iter; 1.2802x vs baseline; 1.2802x over previous
import jax
import jax.numpy as jnp
from jax import lax
from jax.experimental import pallas as pl
from jax.experimental.pallas import tpu as pltpu

M, D = 512, 512
HALF = M // 2
C = 8
ROWS = HALF // C


def kernel(partial, resid, gamma):
    def body(
        partial_ref, resid_ref, gamma_ref, out_ref, recv_ref,
        send_a_sems, recv_a_sems, send_b_sems, recv_b_sems,
    ):
        my_x = lax.axis_index("x")
        my_y = lax.axis_index("y")
        y_nbr = (my_x, 1 - my_y)
        x_nbr = (1 - my_x, my_y)

        barrier = pltpu.get_barrier_semaphore()
        for nbr in (y_nbr, x_nbr):
            pl.semaphore_signal(
                barrier, inc=1, device_id=nbr,
                device_id_type=pl.DeviceIdType.MESH,
            )
        pl.semaphore_wait(barrier, 2)

        row0 = my_x * HALF
        gamma_row = gamma_ref[0, :][None, :]

        rdma_a = []
        for k in range(C):
            r = pltpu.make_async_remote_copy(
                src_ref=partial_ref.at[0, pl.ds(row0 + k * ROWS, ROWS), :],
                dst_ref=recv_ref.at[pl.ds(k * ROWS, ROWS), :],
                send_sem=send_a_sems.at[k],
                recv_sem=recv_a_sems.at[k],
                device_id=y_nbr,
                device_id_type=pl.DeviceIdType.MESH,
            )
            r.start()
            rdma_a.append(r)

        rdma_b = []
        for k in range(C):
            rdma_a[k].wait_recv()
            sl = pl.ds(row0 + k * ROWS, ROWS)
            y = (
                partial_ref[0, sl, :]
                + recv_ref[pl.ds(k * ROWS, ROWS), :]
                + resid_ref[sl, :]
            )
            rms = jnp.sqrt(jnp.mean(y * y, axis=-1, keepdims=True) + 1e-6)
            out_ref[sl, :] = y / rms * gamma_row
            rb = pltpu.make_async_remote_copy(
                src_ref=out_ref.at[sl, :],
                dst_ref=out_ref.at[sl, :],
                send_sem=send_b_sems.at[k],
                recv_sem=recv_b_sems.at[k],
                device_id=x_nbr,
                device_id_type=pl.DeviceIdType.MESH,
            )
            rb.start()
            rdma_b.append(rb)

        for k in range(C):
            rdma_a[k].wait_send()
            rdma_b[k].wait()

    return pl.pallas_call(
        body,
        out_shape=jax.ShapeDtypeStruct((M, D), jnp.float32),
        in_specs=[
            pl.BlockSpec(memory_space=pltpu.VMEM),
            pl.BlockSpec(memory_space=pltpu.VMEM),
            pl.BlockSpec(memory_space=pltpu.VMEM),
        ],
        out_specs=pl.BlockSpec(memory_space=pltpu.VMEM),
        scratch_shapes=[
            pltpu.VMEM((HALF, D), jnp.float32),
            pltpu.SemaphoreType.DMA((C,)),
            pltpu.SemaphoreType.DMA((C,)),
            pltpu.SemaphoreType.DMA((C,)),
            pltpu.SemaphoreType.DMA((C,)),
        ],
        compiler_params=pltpu.CompilerParams(collective_id=0),
    )(partial, resid, gamma.reshape(1, D))


# device time: 13851 ns/iter; 1.5631x vs baseline; 1.2210x over previous
import jax
import jax.numpy as jnp
from jax import lax
from jax.experimental import pallas as pl
from jax.experimental.pallas import tpu as pltpu

M, D = 512, 512
HALF = M // 2
C = 8
ROWS = HALF // C


def kernel(partial, resid, gamma):
    def body(
        partial_ref, resid_ref, gamma_ref, out_ref,
        send_a, recv_a, send_b, recv_b,
        send_a_sems, recv_a_sems, send_b_sems, recv_b_sems,
    ):
        my_x = lax.axis_index("x")
        my_y = lax.axis_index("y")
        y_nbr = (my_x, 1 - my_y)
        x_nbr = (1 - my_x, my_y)

        barrier = pltpu.get_barrier_semaphore()
        for nbr in (y_nbr, x_nbr):
            pl.semaphore_signal(
                barrier, inc=1, device_id=nbr,
                device_id_type=pl.DeviceIdType.MESH,
            )
        pl.semaphore_wait(barrier, 2)

        row0 = my_x * HALF
        other0 = (1 - my_x) * HALF
        gamma_row = gamma_ref[0, :][None, :]

        rdma_a = []
        for k in range(C):
            ck = pl.ds(k * ROWS, ROWS)
            send_a[ck, :] = partial_ref[0, pl.ds(row0 + k * ROWS, ROWS), :].astype(
                jnp.bfloat16
            )
            r = pltpu.make_async_remote_copy(
                src_ref=send_a.at[ck, :],
                dst_ref=recv_a.at[ck, :],
                send_sem=send_a_sems.at[k],
                recv_sem=recv_a_sems.at[k],
                device_id=y_nbr,
                device_id_type=pl.DeviceIdType.MESH,
            )
            r.start()
            rdma_a.append(r)

        rdma_b = []
        for k in range(C):
            rdma_a[k].wait_recv()
            ck = pl.ds(k * ROWS, ROWS)
            sl = pl.ds(row0 + k * ROWS, ROWS)
            y = (
                partial_ref[0, sl, :]
                + recv_a[ck, :].astype(jnp.float32)
                + resid_ref[sl, :]
            )
            rms = jnp.sqrt(jnp.mean(y * y, axis=-1, keepdims=True) + 1e-6)
            o = y / rms * gamma_row
            out_ref[sl, :] = o
            send_b[ck, :] = o.astype(jnp.bfloat16)
            rb = pltpu.make_async_remote_copy(
                src_ref=send_b.at[ck, :],
                dst_ref=recv_b.at[ck, :],
                send_sem=send_b_sems.at[k],
                recv_sem=recv_b_sems.at[k],
                device_id=x_nbr,
                device_id_type=pl.DeviceIdType.MESH,
            )
            rb.start()
            rdma_b.append(rb)

        for k in range(C):
            rdma_b[k].wait_recv()
            ck = pl.ds(k * ROWS, ROWS)
            out_ref[pl.ds(other0 + k * ROWS, ROWS), :] = recv_b[ck, :].astype(
                jnp.float32
            )
        for k in range(C):
            rdma_a[k].wait_send()
            rdma_b[k].wait_send()

    return pl.pallas_call(
        body,
        out_shape=jax.ShapeDtypeStruct((M, D), jnp.float32),
        in_specs=[
            pl.BlockSpec(memory_space=pltpu.VMEM),
            pl.BlockSpec(memory_space=pltpu.VMEM),
            pl.BlockSpec(memory_space=pltpu.VMEM),
        ],
        out_specs=pl.BlockSpec(memory_space=pltpu.VMEM),
        scratch_shapes=[
            pltpu.VMEM((HALF, D), jnp.bfloat16),
            pltpu.VMEM((HALF, D), jnp.bfloat16),
            pltpu.VMEM((HALF, D), jnp.bfloat16),
            pltpu.VMEM((HALF, D), jnp.bfloat16),
            pltpu.SemaphoreType.DMA((C,)),
            pltpu.SemaphoreType.DMA((C,)),
            pltpu.SemaphoreType.DMA((C,)),
            pltpu.SemaphoreType.DMA((C,)),
        ],
        compiler_params=pltpu.CompilerParams(collective_id=0),
    )(partial, resid, gamma.reshape(1, D))


# device time: 13316 ns/iter; 1.6259x vs baseline; 1.0402x over previous
import jax
import jax.numpy as jnp
from jax import lax
from jax.experimental import pallas as pl
from jax.experimental.pallas import tpu as pltpu

M, D = 512, 512
HALF = M // 2
ROWS = 32
EXTRA = 96
C_A = (HALF + EXTRA) // ROWS
C_B = (HALF - EXTRA) // ROWS


def kernel(partial, resid, gamma):
    def body(
        partial_ref, resid_ref, gamma_ref, out_ref,
        send_a, recv_a, send_b, recv_b,
        send_a_sems, recv_a_sems, send_b_sems, recv_b_sems,
    ):
        my_x = lax.axis_index("x")
        my_y = lax.axis_index("y")
        y_nbr = (my_x, 1 - my_y)
        x_nbr = (1 - my_x, my_y)

        base = my_x * HALF
        other = (1 - my_x) * HALF

        def a_row(k):
            if k < C_B:
                return base + EXTRA + ROWS * k
            if k < C_B + EXTRA // ROWS:
                return base + ROWS * (k - C_B)
            return other + ROWS * (k - C_B - EXTRA // ROWS)

        barrier = pltpu.get_barrier_semaphore()
        for nbr in (y_nbr, x_nbr):
            pl.semaphore_signal(
                barrier, inc=1, device_id=nbr,
                device_id_type=pl.DeviceIdType.MESH,
            )
        for k in range(C_A):
            send_a[pl.ds(ROWS * k, ROWS), :] = partial_ref[
                0, pl.ds(a_row(k), ROWS), :
            ].astype(jnp.bfloat16)
        pl.semaphore_wait(barrier, 2)

        rdma_a = []
        for k in range(C_A):
            r = pltpu.make_async_remote_copy(
                src_ref=send_a.at[pl.ds(ROWS * k, ROWS), :],
                dst_ref=recv_a.at[pl.ds(ROWS * k, ROWS), :],
                send_sem=send_a_sems.at[k],
                recv_sem=recv_a_sems.at[k],
                device_id=y_nbr,
                device_id_type=pl.DeviceIdType.MESH,
            )
            r.start()
            rdma_a.append(r)

        gamma_row = gamma_ref[0, :][None, :]

        rdma_b = []
        for k in range(C_A):
            rdma_a[k].wait_recv()
            ck = pl.ds(ROWS * k, ROWS)
            sl = pl.ds(a_row(k), ROWS)
            y = (
                partial_ref[0, sl, :]
                + recv_a[ck, :].astype(jnp.float32)
                + resid_ref[sl, :]
            )
            rms = jnp.sqrt(jnp.mean(y * y, axis=-1, keepdims=True) + 1e-6)
            o = y / rms * gamma_row
            out_ref[sl, :] = o
            if k < C_B:
                send_b[ck, :] = o.astype(jnp.bfloat16)
                rb = pltpu.make_async_remote_copy(
                    src_ref=send_b.at[ck, :],
                    dst_ref=recv_b.at[ck, :],
                    send_sem=send_b_sems.at[k],
                    recv_sem=recv_b_sems.at[k],
                    device_id=x_nbr,
                    device_id_type=pl.DeviceIdType.MESH,
                )
                rb.start()
                rdma_b.append(rb)

        for j in range(C_B):
            rdma_b[j].wait_recv()
            out_ref[pl.ds(other + EXTRA + ROWS * j, ROWS), :] = recv_b[
                pl.ds(ROWS * j, ROWS), :
            ].astype(jnp.float32)
        for k in range(C_A):
            rdma_a[k].wait_send()
        for j in range(C_B):
            rdma_b[j].wait_send()

    return pl.pallas_call(
        body,
        out_shape=jax.ShapeDtypeStruct((M, D), jnp.float32),
        in_specs=[
            pl.BlockSpec(memory_space=pltpu.VMEM),
            pl.BlockSpec(memory_space=pltpu.VMEM),
            pl.BlockSpec(memory_space=pltpu.VMEM),
        ],
        out_specs=pl.BlockSpec(memory_space=pltpu.VMEM),
        scratch_shapes=[
            pltpu.VMEM((HALF + EXTRA, D), jnp.bfloat16),
            pltpu.VMEM((HALF + EXTRA, D), jnp.bfloat16),
            pltpu.VMEM((HALF - EXTRA, D), jnp.bfloat16),
            pltpu.VMEM((HALF - EXTRA, D), jnp.bfloat16),
            pltpu.SemaphoreType.DMA((C_A,)),
            pltpu.SemaphoreType.DMA((C_A,)),
            pltpu.SemaphoreType.DMA((C_B,)),
            pltpu.SemaphoreType.DMA((C_B,)),
        ],
        compiler_params=pltpu.CompilerParams(collective_id=0),
    )(partial, resid, gamma.reshape(1, D))


# device time: 13253 ns/iter; 1.6337x vs baseline; 1.0048x over previous
import jax
import jax.numpy as jnp
from jax import lax
from jax.experimental import pallas as pl
from jax.experimental.pallas import tpu as pltpu

M, D = 512, 512
HALF = M // 2
ROWS = 32
EXTRA = 96
PRIO = HALF - EXTRA
C_B = PRIO // ROWS
N_A = C_B + 2


def kernel(partial, resid, gamma):
    def body(
        partial_ref, resid_ref, gamma_ref, out_ref,
        send_a, recv_a, send_b, recv_b,
        send_a_sems, recv_a_sems, send_b_sems, recv_b_sems,
    ):
        my_x = lax.axis_index("x")
        my_y = lax.axis_index("y")
        y_nbr = (my_x, 1 - my_y)
        x_nbr = (1 - my_x, my_y)

        base = my_x * HALF
        other = (1 - my_x) * HALF

        a_regions = [(ROWS * k, base + EXTRA + ROWS * k, ROWS) for k in range(C_B)]
        a_regions.append((PRIO, base, EXTRA))
        a_regions.append((PRIO + EXTRA, other, EXTRA))

        barrier = pltpu.get_barrier_semaphore()
        for nbr in (y_nbr, x_nbr):
            pl.semaphore_signal(
                barrier, inc=1, device_id=nbr,
                device_id_type=pl.DeviceIdType.MESH,
            )
        send_a[pl.ds(0, PRIO), :] = partial_ref[
            0, pl.ds(base + EXTRA, PRIO), :
        ].astype(jnp.bfloat16)
        send_a[pl.ds(PRIO, EXTRA), :] = partial_ref[0, pl.ds(base, EXTRA), :].astype(
            jnp.bfloat16
        )
        send_a[pl.ds(PRIO + EXTRA, EXTRA), :] = partial_ref[
            0, pl.ds(other, EXTRA), :
        ].astype(jnp.bfloat16)
        pl.semaphore_wait(barrier, 2)

        rdma_a = []
        for k, (boff, _, n) in enumerate(a_regions):
            r = pltpu.make_async_remote_copy(
                src_ref=send_a.at[pl.ds(boff, n), :],
                dst_ref=recv_a.at[pl.ds(boff, n), :],
                send_sem=send_a_sems.at[k],
                recv_sem=recv_a_sems.at[k],
                device_id=y_nbr,
                device_id_type=pl.DeviceIdType.MESH,
            )
            r.start()
            rdma_a.append(r)

        gamma_row = gamma_ref[0, :][None, :]

        def reduce_norm(boff, roff, n):
            y = (
                partial_ref[0, pl.ds(roff, n), :]
                + recv_a[pl.ds(boff, n), :].astype(jnp.float32)
                + resid_ref[pl.ds(roff, n), :]
            )
            rms = jnp.sqrt(jnp.mean(y * y, axis=-1, keepdims=True) + 1e-6)
            o = y / rms * gamma_row
            out_ref[pl.ds(roff, n), :] = o
            return o

        rdma_b = []
        for k in range(C_B):
            rdma_a[k].wait_recv()
            boff, roff, n = a_regions[k]
            o = reduce_norm(boff, roff, n)
            send_b[pl.ds(boff, n), :] = o.astype(jnp.bfloat16)
            rb = pltpu.make_async_remote_copy(
                src_ref=send_b.at[pl.ds(boff, n), :],
                dst_ref=recv_b.at[pl.ds(boff, n), :],
                send_sem=send_b_sems.at[k],
                recv_sem=recv_b_sems.at[k],
                device_id=x_nbr,
                device_id_type=pl.DeviceIdType.MESH,
            )
            rb.start()
            rdma_b.append(rb)

        for k in (C_B, C_B + 1):
            rdma_a[k].wait_recv()
            boff, roff, n = a_regions[k]
            reduce_norm(boff, roff, n)

        for j in range(C_B):
            rdma_b[j].wait_recv()
        out_ref[pl.ds(other + EXTRA, PRIO), :] = recv_b[:, :].astype(jnp.float32)

        for k in range(N_A):
            rdma_a[k].wait_send()
        for j in range(C_B):
            rdma_b[j].wait_send()

    return pl.pallas_call(
        body,
        out_shape=jax.ShapeDtypeStruct((M, D), jnp.float32),
        in_specs=[
            pl.BlockSpec(memory_space=pltpu.VMEM),
            pl.BlockSpec(memory_space=pltpu.VMEM),
            pl.BlockSpec(memory_space=pltpu.VMEM),
        ],
        out_specs=pl.BlockSpec(memory_space=pltpu.VMEM),
        scratch_shapes=[
            pltpu.VMEM((HALF + EXTRA, D), jnp.bfloat16),
            pltpu.VMEM((HALF + EXTRA, D), jnp.bfloat16),
            pltpu.VMEM((PRIO, D), jnp.bfloat16),
            pltpu.VMEM((PRIO, D), jnp.bfloat16),
            pltpu.SemaphoreType.DMA((N_A,)),
            pltpu.SemaphoreType.DMA((N_A,)),
            pltpu.SemaphoreType.DMA((C_B,)),
            pltpu.SemaphoreType.DMA((C_B,)),
        ],
        compiler_params=pltpu.CompilerParams(collective_id=0),
    )(partial, resid, gamma.reshape(1, D))
